# baseline (speedup 1.0000x reference)
import functools

import jax
import jax.numpy as jnp
from jax import lax
from jax.experimental import pallas as pl
from jax.experimental.pallas import tpu as pltpu
from jax.experimental.pallas import tpu_sc as plsc

_TOP_K = 8
_NEG_INF = float("-inf")


def _matmul_kernel(x_ref, w_ref, logits_ref):
    logits_ref[...] = jax.lax.dot_general(
        w_ref[...], x_ref[...], (((1,), (1,)), ((), ())),
        preferred_element_type=jnp.float32)


def _tc_matmul(x, W):
    n_tokens, d_model = x.shape
    n_experts = W.shape[0]
    block_r = 1024
    n_blocks = n_tokens // block_r
    return pl.pallas_call(
        _matmul_kernel,
        grid=(n_blocks,),
        in_specs=[
            pl.BlockSpec((block_r, d_model), lambda i: (i, 0)),
            pl.BlockSpec((n_experts, d_model), lambda i: (0, 0)),
        ],
        out_specs=pl.BlockSpec((n_experts, block_r), lambda i: (0, i)),
        out_shape=jax.ShapeDtypeStruct((n_experts, n_tokens), jnp.float32),
    )(x, W)


def _sc_router(logits_t, n_experts, n_tokens):
    info = plsc.get_sparse_core_info()
    nc, ns, nl = info.num_cores, info.num_subcores, info.num_lanes
    nw = nc * ns
    tok_per_w = n_tokens // nw
    chunk = 512
    n_chunks = tok_per_w // chunk
    groups = chunk // nl
    mesh = plsc.VectorSubcoreMesh(core_axis_name="c", subcore_axis_name="s")

    @functools.partial(
        pl.kernel, mesh=mesh,
        out_type=[
            jax.ShapeDtypeStruct((n_experts, n_tokens), jnp.float32),
            jax.ShapeDtypeStruct((_TOP_K, n_tokens), jnp.int32),
        ],
        scratch_types=[
            pltpu.VMEM((n_experts, chunk), jnp.float32),
            pltpu.VMEM((n_experts, chunk), jnp.float32),
            pltpu.VMEM((_TOP_K, chunk), jnp.int32),
        ],
    )
    def sc_kernel(logits_hbm, gates_hbm, idx_hbm, lbuf, gbuf, ibuf):
        wid = lax.axis_index("s") * nc + lax.axis_index("c")
        for c in range(n_chunks):
            base = wid * tok_per_w + c * chunk
            pltpu.sync_copy(logits_hbm.at[:, pl.ds(base, chunk)], lbuf)

            def group_body(g, carry):
                col = g * nl
                vecs = [lbuf[e, pl.ds(col, nl)] for e in range(n_experts)]
                work = list(vecs)
                top1 = None
                for k in range(_TOP_K):
                    m = work[0]
                    for e in range(1, n_experts):
                        m = jnp.maximum(m, work[e])
                    if top1 is None:
                        top1 = m
                    amin = jnp.where(work[0] == m, 0, n_experts)
                    for e in range(1, n_experts):
                        cand = jnp.where(work[e] == m, e, n_experts)
                        amin = jnp.minimum(amin, cand)
                    ibuf[k, pl.ds(col, nl)] = amin
                    for e in range(n_experts):
                        work[e] = jnp.where(amin == e, _NEG_INF, work[e])
                gates = [jnp.where(work[e] == _NEG_INF,
                                   jnp.exp(vecs[e] - top1), 0.0)
                         for e in range(n_experts)]
                denom = gates[0]
                for e in range(1, n_experts):
                    denom = denom + gates[e]
                inv = 1.0 / denom
                for e in range(n_experts):
                    gbuf[e, pl.ds(col, nl)] = gates[e] * inv
                return carry

            lax.fori_loop(0, groups, group_body, 0)
            pltpu.sync_copy(gbuf, gates_hbm.at[:, pl.ds(base, chunk)])
            pltpu.sync_copy(ibuf, idx_hbm.at[:, pl.ds(base, chunk)])

    return sc_kernel(logits_t)


def _finalize_kernel(gt_ref, it_ref, gates_ref, idx_ref, loss_ref,
                     gsum_ref, cnt_ref, *, n_tokens, n_blocks, n_experts):
    i = pl.program_id(0)
    gt = gt_ref[...]
    gates_ref[...] = gt.T
    idx_ref[...] = it_ref[...].T
    part_g = jnp.sum(gt, axis=1, keepdims=True)
    part_c = jnp.sum((gt > 0.0).astype(jnp.float32), axis=1, keepdims=True)

    @pl.when(i == 0)
    def _init():
        gsum_ref[...] = jnp.zeros_like(gsum_ref)
        cnt_ref[...] = jnp.zeros_like(cnt_ref)

    gsum_ref[...] += part_g
    cnt_ref[...] += part_c

    @pl.when(i == n_blocks - 1)
    def _finalize():
        inv_n = 1.0 / float(n_tokens)
        loss = jnp.sum(gsum_ref[...] * inv_n * cnt_ref[...] * inv_n)
        loss_ref[0, 0] = loss * float(n_experts)


def _tc_finalize(gates_t, idx_t, n_experts, n_tokens):
    block_r = 2048
    n_blocks = n_tokens // block_r
    grid_spec = pltpu.PrefetchScalarGridSpec(
        num_scalar_prefetch=0,
        grid=(n_blocks,),
        in_specs=[
            pl.BlockSpec((n_experts, block_r), lambda i: (0, i)),
            pl.BlockSpec((_TOP_K, block_r), lambda i: (0, i)),
        ],
        out_specs=[
            pl.BlockSpec((block_r, n_experts), lambda i: (i, 0)),
            pl.BlockSpec((block_r, _TOP_K), lambda i: (i, 0)),
            pl.BlockSpec(memory_space=pltpu.SMEM),
        ],
        scratch_shapes=[
            pltpu.VMEM((n_experts, 1), jnp.float32),
            pltpu.VMEM((n_experts, 1), jnp.float32),
        ],
    )
    return pl.pallas_call(
        functools.partial(_finalize_kernel, n_tokens=n_tokens,
                          n_blocks=n_blocks, n_experts=n_experts),
        grid_spec=grid_spec,
        out_shape=[
            jax.ShapeDtypeStruct((n_tokens, n_experts), jnp.float32),
            jax.ShapeDtypeStruct((n_tokens, _TOP_K), jnp.int32),
            jax.ShapeDtypeStruct((1, 1), jnp.float32),
        ],
        compiler_params=pltpu.CompilerParams(
            dimension_semantics=("arbitrary",),
        ),
    )(gates_t, idx_t)


def kernel(x, W):
    n_tokens = x.shape[0]
    n_experts = W.shape[0]
    logits_t = _tc_matmul(x, W)
    gates_t, idx_t = _sc_router(logits_t, n_experts, n_tokens)
    gates, idx, loss = _tc_finalize(gates_t, idx_t, n_experts, n_tokens)
    return gates, idx, loss[0, 0]


# speedup vs baseline: 1.4713x; 1.4713x over previous
import functools

import jax
import jax.numpy as jnp
from jax.experimental import pallas as pl
from jax.experimental.pallas import tpu as pltpu

_TOP_K = 8
_NEG_INF = float("-inf")


def _router_kernel(x_ref, w_ref, gates_ref, idx_ref, loss_ref,
                   gsum_ref, cnt_ref, *, n_tokens, n_blocks, n_experts):
    i = pl.program_id(0)
    x = x_ref[...]
    w = w_ref[...]
    logits_t = jax.lax.dot_general(
        w, x, (((1,), (1,)), ((), ())),
        preferred_element_type=jnp.float32)

    r = logits_t.shape[1]
    e_iota = jax.lax.broadcasted_iota(jnp.int32, (n_experts, r), 0)

    work = logits_t
    idx_rows = []
    top1 = None
    for k in range(_TOP_K):
        m = jnp.max(work, axis=0, keepdims=True)
        if top1 is None:
            top1 = m
        amax = jnp.min(jnp.where(work == m, e_iota, n_experts),
                       axis=0, keepdims=True)
        idx_rows.append(amax)
        work = jnp.where(e_iota == amax, _NEG_INF, work)

    sel = work == _NEG_INF
    e = jnp.where(sel, jnp.exp(logits_t - top1), 0.0)
    denom = jnp.sum(e, axis=0, keepdims=True)
    gates_t = e / denom
    gates_ref[...] = gates_t.T
    idx_ref[...] = jnp.concatenate(idx_rows, axis=0).T

    part_g = jnp.sum(gates_t, axis=1, keepdims=True)
    part_c = jnp.sum(sel.astype(jnp.float32), axis=1, keepdims=True)

    @pl.when(i == 0)
    def _init():
        gsum_ref[...] = jnp.zeros_like(gsum_ref)
        cnt_ref[...] = jnp.zeros_like(cnt_ref)

    gsum_ref[...] += part_g
    cnt_ref[...] += part_c

    @pl.when(i == n_blocks - 1)
    def _finalize():
        inv_n = 1.0 / float(n_tokens)
        loss = jnp.sum(gsum_ref[...] * inv_n * cnt_ref[...] * inv_n)
        loss_ref[0, 0] = loss * float(n_experts)


def kernel(x, W):
    n_tokens, d_model = x.shape
    n_experts = W.shape[0]
    block_r = 1024
    n_blocks = n_tokens // block_r

    grid_spec = pltpu.PrefetchScalarGridSpec(
        num_scalar_prefetch=0,
        grid=(n_blocks,),
        in_specs=[
            pl.BlockSpec((block_r, d_model), lambda i: (i, 0)),
            pl.BlockSpec((n_experts, d_model), lambda i: (0, 0)),
        ],
        out_specs=[
            pl.BlockSpec((block_r, n_experts), lambda i: (i, 0)),
            pl.BlockSpec((block_r, _TOP_K), lambda i: (i, 0)),
            pl.BlockSpec(memory_space=pltpu.SMEM),
        ],
        scratch_shapes=[
            pltpu.VMEM((n_experts, 1), jnp.float32),
            pltpu.VMEM((n_experts, 1), jnp.float32),
        ],
    )

    gates, idx, loss = pl.pallas_call(
        functools.partial(_router_kernel, n_tokens=n_tokens,
                          n_blocks=n_blocks, n_experts=n_experts),
        grid_spec=grid_spec,
        out_shape=[
            jax.ShapeDtypeStruct((n_tokens, n_experts), jnp.float32),
            jax.ShapeDtypeStruct((n_tokens, _TOP_K), jnp.int32),
            jax.ShapeDtypeStruct((1, 1), jnp.float32),
        ],
        compiler_params=pltpu.CompilerParams(
            dimension_semantics=("arbitrary",),
        ),
    )(x, W)
    return gates, idx, loss[0, 0]
